# baseline (speedup 1.0000x reference)
import functools

import jax
import jax.numpy as jnp
from jax import lax
from jax.experimental import pallas as pl
from jax.experimental.pallas import tpu as pltpu
from jax.experimental.pallas import tpu_sc as plsc


def _lane_allsum(v):
    idx = lax.iota(jnp.int32, 16)
    for sh in (8, 4, 2, 1):
        v = v + v.at[idx ^ sh].get(mode="promise_in_bounds")
    return v


L = 16
NG = 5
ROWS = 77
RPAD = 80
NO = 90
XTB = 0
WB = RPAD * 80
AX = WB + NG * NO * L


def _body(pack_hbm, out_hbm, xt_v, wb_v, ax_v, est_v, sem):
    tid = lax.axis_index("s")

    @pl.when(tid < NG)
    def _work():
        cp1 = pltpu.async_copy(
            pack_hbm.at[pl.ds(XTB + tid * (80 * L), 80 * L)], xt_v, sem)
        cp2 = pltpu.async_copy(pack_hbm.at[pl.ds(WB, NG * NO * L)], wb_v, sem)
        cp3 = pltpu.async_copy(
            pack_hbm.at[pl.ds(AX, 2 * NO + RPAD)], ax_v, sem)
        cp1.wait()
        cp2.wait()
        cp3.wait()

        acc1 = [jnp.zeros((L,), jnp.float32) for _ in range(NG)]
        acc2 = [jnp.zeros((L,), jnp.float32) for _ in range(NG)]
        for blk in range(6):
            coefs1 = ax_v[pl.ds(blk * L, L)]
            coefs2 = ax_v[pl.ds(NO + blk * L, L)]
            for l in range(L):
                o = blk * L + l
                if o >= NO:
                    break
                c1 = coefs1[l]
                c2 = coefs2[l]
                for g in range(NG):
                    wrow = wb_v[pl.ds((g * NO + o) * L, L)]
                    acc1[g] = acc1[g] + c1 * wrow
                    acc2[g] = acc2[g] + c2 * wrow

        sacc = jnp.zeros((L,), jnp.float32)
        for g in range(NG):
            sacc = sacc + ax_v[pl.ds(2 * NO + g * L, L)] * acc1[g]
        s_vec = _lane_allsum(sacc)

        acc = jnp.zeros((L,), jnp.float32)
        for blk in range(NG):
            for l in range(L):
                k = blk * L + l
                acc = acc + acc2[blk][l] * xt_v[pl.ds(k * L, L)]
        t = acc + s_vec
        est_v[...] = jnp.where(t >= 0.0, t, 0.2 * t)
        pltpu.sync_copy(est_v, out_hbm.at[pl.ds(tid * L, L)])


@functools.partial(
    pl.kernel,
    out_type=jax.ShapeDtypeStruct((RPAD,), jnp.float32),
    mesh=plsc.VectorSubcoreMesh(core_axis_name="c", subcore_axis_name="s",
                                num_cores=1),
    scratch_types=[
        pltpu.VMEM((80 * L,), jnp.float32),
        pltpu.VMEM((NG * NO * L,), jnp.float32),
        pltpu.VMEM((2 * NO + RPAD,), jnp.float32),
        pltpu.VMEM((L,), jnp.float32),
        pltpu.SemaphoreType.DMA,
    ],
    compiler_params=pltpu.CompilerParams(needs_layout_passes=False),
    name="similarity_sc",
)
def _similarity_sc(pack_hbm, out_hbm, *scratch):
    _body(pack_hbm, out_hbm, *scratch)


def kernel(chicago_region_representations, W_lin, W_attn):
    x = jnp.asarray(chicago_region_representations, jnp.float32)
    xp = jnp.zeros((RPAD, 80), jnp.float32).at[:ROWS].set(x)
    xtb = xp.T.reshape(80, NG, L).transpose(1, 0, 2)
    wb = W_lin.astype(jnp.float32).reshape(NO, NG, L).transpose(1, 0, 2)
    pack = jnp.concatenate([
        xtb.reshape(-1),
        wb.reshape(-1),
        W_attn.astype(jnp.float32).reshape(-1),
        x[10],
    ])
    e = _similarity_sc(pack)
    return e[:ROWS].reshape(ROWS, 1)
